# Initial kernel scaffold; baseline (speedup 1.0000x reference)
import jax
import jax.numpy as jnp
from jax.experimental import pallas as pl


def kernel(x_in, h_node, batch, W_gnn, b_gnn, W_gate, b_gate, g):
    raise NotImplementedError("write your pallas kernel here")


# speedup vs baseline: 4.4559x; 4.4559x over previous
import functools

import jax
import jax.numpy as jnp
from jax.experimental import pallas as pl
from jax.experimental.pallas import tpu as pltpu

N = 100000
D = 128
G = 512
B = 2000
NB = N // B


def _tc_body(x_ref, h_ref, b_ref, g_ref, wg_ref, bg_ref, wgate_ref, bgate_ref,
             hout_ref, cout_ref, r_ref, env_ref, gate_ref,
             acc_h, acc_gh, acc_c):
    i = pl.program_id(0)

    @pl.when(i == 0)
    def _init():
        acc_h[...] = jnp.zeros_like(acc_h)
        acc_gh[...] = jnp.zeros_like(acc_gh)
        acc_c[...] = jnp.zeros_like(acc_c)

    x = x_ref[...]
    h = h_ref[...]
    z = jax.lax.dot(x, wg_ref[...], preferred_element_type=jnp.float32)
    z = z + bg_ref[...]

    wgate = wgate_ref[...]
    wd = wgate[:, 1:2] - wgate[:, 0:1]
    bd = bgate_ref[0, 1] - bgate_ref[0, 0]
    gd = g_ref[:, 1:2] - g_ref[:, 0:1]
    logit = jax.lax.dot(z, wd, preferred_element_type=jnp.float32) + bd + gd
    gate = jax.nn.sigmoid(logit)
    gate_ref[...] = gate

    ids = b_ref[0, 0, :]
    seg = jax.lax.broadcasted_iota(jnp.int32, (B, G), 1)
    oh = (ids[:, None] == seg).astype(jnp.float32)

    vals = jnp.concatenate([h, gate * h], axis=1)
    contrib = jax.lax.dot_general(
        oh, vals, (((0,), (0,)), ((), ())),
        preferred_element_type=jnp.float32)
    acc_h[...] += contrib[:, :D]
    acc_gh[...] += contrib[:, D:]

    ones = jnp.ones_like(gate)
    cvals = jnp.concatenate(
        [ones, gate, jnp.zeros((B, 6), jnp.float32)], axis=1)
    ccontrib = jax.lax.dot_general(
        oh, cvals, (((0,), (0,)), ((), ())),
        preferred_element_type=jnp.float32)
    acc_c[...] += ccontrib

    @pl.when(i == NB - 1)
    def _finalize():
        count = acc_c[:, 0:1]
        sgate = acc_c[:, 1:2]
        cc = jnp.maximum(count, 1.0)
        hout_ref[...] = acc_gh[...] / cc
        cout_ref[...] = (acc_h[...] - acc_gh[...]) / cc
        r_ref[...] = sgate + 1e-8
        env_ref[...] = count - sgate + 1e-8


@functools.partial(jax.jit, static_argnames=())
def kernel(x_in, h_node, batch, W_gnn, b_gnn, W_gate, b_gate, g):
    batch_i32 = batch.astype(jnp.int32).reshape(NB, 1, B)
    b_gnn2 = b_gnn.reshape(1, D)
    b_gate2 = b_gate.reshape(1, 2)

    out_shapes = (
        jax.ShapeDtypeStruct((G, D), jnp.float32),
        jax.ShapeDtypeStruct((G, D), jnp.float32),
        jax.ShapeDtypeStruct((G, 1), jnp.float32),
        jax.ShapeDtypeStruct((G, 1), jnp.float32),
        jax.ShapeDtypeStruct((N, 1), jnp.float32),
    )
    grid = (NB,)
    in_specs = [
        pl.BlockSpec((B, D), lambda i: (i, 0)),
        pl.BlockSpec((B, D), lambda i: (i, 0)),
        pl.BlockSpec((1, 1, B), lambda i: (i, 0, 0)),
        pl.BlockSpec((B, 2), lambda i: (i, 0)),
        pl.BlockSpec((D, D), lambda i: (0, 0)),
        pl.BlockSpec((1, D), lambda i: (0, 0)),
        pl.BlockSpec((D, 2), lambda i: (0, 0)),
        pl.BlockSpec((1, 2), lambda i: (0, 0)),
    ]
    out_specs = (
        pl.BlockSpec((G, D), lambda i: (0, 0)),
        pl.BlockSpec((G, D), lambda i: (0, 0)),
        pl.BlockSpec((G, 1), lambda i: (0, 0)),
        pl.BlockSpec((G, 1), lambda i: (0, 0)),
        pl.BlockSpec((B, 1), lambda i: (i, 0)),
    )
    scratch = [
        pltpu.VMEM((G, D), jnp.float32),
        pltpu.VMEM((G, D), jnp.float32),
        pltpu.VMEM((G, 8), jnp.float32),
    ]
    h_out, c_out, r_node_num, env_node_num, gate = pl.pallas_call(
        _tc_body,
        grid=grid,
        in_specs=in_specs,
        out_specs=out_specs,
        out_shape=out_shapes,
        scratch_shapes=scratch,
    )(x_in, h_node, batch_i32, g, W_gnn, b_gnn2, W_gate, b_gate2)
    return (h_out, c_out, r_node_num, env_node_num, gate)
